# Initial kernel scaffold; baseline (speedup 1.0000x reference)
import functools

import jax
import jax.numpy as jnp
from jax import lax
from jax.experimental import pallas as pl
from jax.experimental.pallas import tpu as pltpu
from jax.experimental.pallas import tpu_sc as plsc

EMBED = 128
ROWS = 4096 * 50
NC, NS = 2, 16
NW = NC * NS
CHUNK = 128
CPW = ROWS // (NW * CHUNK)


def _emb_body(x_hbm, table_hbm, out_hbm, idx_v, rows_v, gsem):
    wid = lax.axis_index("s") * NC + lax.axis_index("c")
    chunk0 = wid * CPW
    pltpu.sync_copy(x_hbm.at[pl.ds(chunk0, CPW)], idx_v)

    def chunk(j, carry):
        pltpu.async_copy(table_hbm.at[idx_v.at[j]], rows_v, gsem).wait()
        pltpu.sync_copy(rows_v, out_hbm.at[pl.ds((chunk0 + j) * CHUNK, CHUNK)])
        return carry

    lax.fori_loop(0, CPW, chunk, 0)


@jax.jit
def _emb(x2, table):
    kern = functools.partial(
        pl.kernel,
        mesh=plsc.VectorSubcoreMesh(core_axis_name="c", subcore_axis_name="s"),
        out_type=jax.ShapeDtypeStruct((ROWS, EMBED), jnp.float32),
        scratch_types=[
            pltpu.VMEM((CPW, CHUNK), jnp.int32),
            pltpu.VMEM((CHUNK, EMBED), jnp.float32),
            pltpu.SemaphoreType.DMA,
        ],
    )(_emb_body)
    return kern(x2, table)


def kernel(x, table):
    x2 = x.reshape(ROWS // CHUNK, CHUNK).astype(jnp.int32)
    out = _emb(x2, table)
    return out.reshape(x.shape[0], x.shape[1], EMBED)


# speedup vs baseline: 3.0711x; 3.0711x over previous
import functools

import jax
import jax.numpy as jnp
from jax import lax
from jax.experimental import pallas as pl
from jax.experimental.pallas import tpu as pltpu
from jax.experimental.pallas import tpu_sc as plsc

EMBED = 128
ROWS = 4096 * 50
NC, NS = 2, 16
NW = NC * NS
CHUNK = 128
CPW = ROWS // (NW * CHUNK)


def _emb_body(x_hbm, table_hbm, out_hbm, idx_v, rows_v, gsem):
    wid = lax.axis_index("s") * NC + lax.axis_index("c")
    chunk0 = wid * CPW
    pltpu.sync_copy(x_hbm.at[wid], idx_v)

    def chunk(j, carry):
        pltpu.async_copy(table_hbm.at[idx_v.at[j]], rows_v, gsem).wait()
        pltpu.sync_copy(rows_v, out_hbm.at[pl.ds((chunk0 + j) * CHUNK, CHUNK)])
        return carry

    lax.fori_loop(0, CPW, chunk, 0)


@jax.jit
def _emb(x2, table):
    kern = functools.partial(
        pl.kernel,
        mesh=plsc.VectorSubcoreMesh(core_axis_name="c", subcore_axis_name="s"),
        out_type=jax.ShapeDtypeStruct((ROWS, EMBED), jnp.float32),
        scratch_types=[
            pltpu.VMEM((CPW, CHUNK), jnp.int32),
            pltpu.VMEM((CHUNK, EMBED), jnp.float32),
            pltpu.SemaphoreType.DMA,
        ],
    )(_emb_body)
    return kern(x2, table)


def kernel(x, table):
    x2 = x.reshape(NW, CPW, CHUNK).astype(jnp.int32)
    out = _emb(x2, table)
    return out.reshape(x.shape[0], x.shape[1], EMBED)


# speedup vs baseline: 3.4789x; 1.1328x over previous
import functools

import jax
import jax.numpy as jnp
from jax import lax
from jax.experimental import pallas as pl
from jax.experimental.pallas import tpu as pltpu
from jax.experimental.pallas import tpu_sc as plsc

EMBED = 128
ROWS = 4096 * 50
NC, NS = 2, 16
NW = NC * NS
CHUNK = 128
CPW = ROWS // (NW * CHUNK)
NBUF = 5
LOOKAHEAD = 3


def _emb_body(x_hbm, table_hbm, out_hbm, idx_v, rows_v, gsem, wsem):
    wid = lax.axis_index("s") * NC + lax.axis_index("c")
    chunk0 = wid * CPW
    pltpu.sync_copy(x_hbm.at[wid], idx_v)

    def g_start(j, b):
        pltpu.async_copy(table_hbm.at[idx_v.at[j]], rows_v.at[b], gsem)

    def g_wait(b):
        pltpu.make_async_copy(table_hbm.at[idx_v.at[0]], rows_v.at[b], gsem).wait()

    def w_start(j, b):
        pltpu.async_copy(
            rows_v.at[b], out_hbm.at[pl.ds((chunk0 + j) * CHUNK, CHUNK)], wsem)

    def w_wait(b):
        pltpu.make_async_copy(
            rows_v.at[b], out_hbm.at[pl.ds(chunk0 * CHUNK, CHUNK)], wsem).wait()

    for i in range(LOOKAHEAD):
        g_start(i, i)

    def group(g, carry):
        for b in range(NBUF):
            j = g * NBUF + b
            jf = j + LOOKAHEAD
            bf = (b + LOOKAHEAD) % NBUF

            @pl.when(jl_and(jf >= NBUF, jf < CPW))
            def _():
                w_wait(bf)

            @pl.when(jf < CPW)
            def _():
                g_start(jf, bf)

            g_wait(b)
            w_start(j, b)
        return carry

    lax.fori_loop(0, CPW // NBUF, group, 0)
    for b in range(NBUF):
        w_wait(b)


def jl_and(a, b):
    return jnp.logical_and(a, b)


@jax.jit
def _emb(x2, table):
    kern = functools.partial(
        pl.kernel,
        mesh=plsc.VectorSubcoreMesh(core_axis_name="c", subcore_axis_name="s"),
        out_type=jax.ShapeDtypeStruct((ROWS, EMBED), jnp.float32),
        scratch_types=[
            pltpu.VMEM((CPW, CHUNK), jnp.int32),
            pltpu.VMEM((NBUF, CHUNK, EMBED), jnp.float32),
            pltpu.SemaphoreType.DMA,
            pltpu.SemaphoreType.DMA,
        ],
    )(_emb_body)
    return kern(x2, table)


def kernel(x, table):
    x2 = x.reshape(NW, CPW, CHUNK).astype(jnp.int32)
    out = _emb(x2, table)
    return out.reshape(x.shape[0], x.shape[1], EMBED)


# speedup vs baseline: 6.2090x; 1.7848x over previous
import functools

import jax
import jax.numpy as jnp
from jax import lax
from jax.experimental import pallas as pl
from jax.experimental.pallas import tpu as pltpu
from jax.experimental.pallas import tpu_sc as plsc

EMBED = 128
BATCH = 4096
SEQ = 50
NC, NS = 2, 16
NW = NC * NS
BPW = BATCH // NW
NBUF = 8
LOOKAHEAD = 4


def _emb_body(x_hbm, table_hbm, out_hbm, idx_v, rows_v, gsem, wsem):
    wid = lax.axis_index("s") * NC + lax.axis_index("c")
    batch0 = wid * BPW
    pltpu.sync_copy(x_hbm.at[wid], idx_v)

    def g_start(j, b):
        pltpu.async_copy(table_hbm.at[idx_v.at[j]], rows_v.at[b], gsem)

    def g_wait(b):
        pltpu.make_async_copy(table_hbm.at[idx_v.at[0]], rows_v.at[b], gsem).wait()

    def w_start(j, b):
        pltpu.async_copy(rows_v.at[b], out_hbm.at[batch0 + j], wsem)

    def w_wait(b):
        pltpu.make_async_copy(rows_v.at[b], out_hbm.at[batch0], wsem).wait()

    for i in range(LOOKAHEAD):
        g_start(i, i)

    def group(g, carry):
        for b in range(NBUF):
            j = g * NBUF + b
            jf = j + LOOKAHEAD
            bf = (b + LOOKAHEAD) % NBUF

            @pl.when(jnp.logical_and(jf >= NBUF, jf < BPW))
            def _():
                w_wait(bf)

            @pl.when(jf < BPW)
            def _():
                g_start(jf, bf)

            g_wait(b)
            w_start(j, b)
        return carry

    lax.fori_loop(0, BPW // NBUF, group, 0)
    for b in range(NBUF):
        w_wait(b)


@jax.jit
def _emb(x2, table):
    kern = functools.partial(
        pl.kernel,
        mesh=plsc.VectorSubcoreMesh(core_axis_name="c", subcore_axis_name="s"),
        out_type=jax.ShapeDtypeStruct((BATCH, SEQ, EMBED), jnp.float32),
        scratch_types=[
            pltpu.VMEM((BPW, SEQ), jnp.int32),
            pltpu.VMEM((NBUF, SEQ, EMBED), jnp.float32),
            pltpu.SemaphoreType.DMA,
            pltpu.SemaphoreType.DMA,
        ],
    )(_emb_body)
    return kern(x2, table)


def kernel(x, table):
    x2 = x.reshape(NW, BPW, SEQ).astype(jnp.int32)
    return _emb(x2, table)


# speedup vs baseline: 11.0580x; 1.7810x over previous
import functools

import jax
import jax.numpy as jnp
from jax import lax
from jax.experimental import pallas as pl
from jax.experimental.pallas import tpu as pltpu
from jax.experimental.pallas import tpu_sc as plsc

EMBED = 128
BATCH = 4096
SEQ = 50
NC, NS = 2, 16
NW = NC * NS
BPW = BATCH // NW
CPW = SEQ
NBUF = 5
LOOKAHEAD = 3


def _emb_body(x_hbm, table_hbm, out_hbm, idx_v, rows_v, gsem, wsem):
    wid = lax.axis_index("s") * NC + lax.axis_index("c")
    batch0 = wid * BPW
    pltpu.sync_copy(x_hbm.at[wid], idx_v)

    def g_start(j, b):
        pltpu.async_copy(table_hbm.at[idx_v.at[j]], rows_v.at[b], gsem)

    def g_wait(b):
        pltpu.make_async_copy(table_hbm.at[idx_v.at[0]], rows_v.at[b], gsem).wait()

    def w_start(j, b):
        pltpu.async_copy(
            rows_v.at[b], out_hbm.at[j].at[pl.ds(batch0, BPW)], wsem)

    def w_wait(b):
        pltpu.make_async_copy(
            rows_v.at[b], out_hbm.at[0].at[pl.ds(batch0, BPW)], wsem).wait()

    for i in range(LOOKAHEAD):
        g_start(i, i)

    def group(g, carry):
        for b in range(NBUF):
            j = g * NBUF + b
            jf = j + LOOKAHEAD
            bf = (b + LOOKAHEAD) % NBUF

            @pl.when(jnp.logical_and(jf >= NBUF, jf < CPW))
            def _():
                w_wait(bf)

            @pl.when(jf < CPW)
            def _():
                g_start(jf, bf)

            g_wait(b)
            w_start(j, b)
        return carry

    lax.fori_loop(0, CPW // NBUF, group, 0)
    for b in range(NBUF):
        w_wait(b)


@jax.jit
def _emb(xw, table):
    kern = functools.partial(
        pl.kernel,
        mesh=plsc.VectorSubcoreMesh(core_axis_name="c", subcore_axis_name="s"),
        out_type=jax.ShapeDtypeStruct((SEQ, BATCH, EMBED), jnp.float32),
        scratch_types=[
            pltpu.VMEM((SEQ, BPW), jnp.int32),
            pltpu.VMEM((NBUF, BPW, EMBED), jnp.float32),
            pltpu.SemaphoreType.DMA,
            pltpu.SemaphoreType.DMA,
        ],
    )(_emb_body)
    out_sbe = kern(xw, table)
    return out_sbe.transpose(1, 0, 2)


def kernel(x, table):
    xw = x.astype(jnp.int32).reshape(NW, BPW, SEQ).transpose(0, 2, 1)
    return _emb(xw, table)
